# baseline (device time: 367933 ns/iter reference)
import numpy as np

import jax
import jax.numpy as jnp
from jax import lax
from jax.experimental import pallas as pl
from jax.experimental.pallas import tpu as pltpu

N_DEV = 32

ORDER = ((0, 0), (2, 0), (1, 0), (3, 0), (0, 1), (2, 1), (1, 1), (3, 1))


def _logical_id(x, y, z):
    row = ((0, 1), (3, 2), (4, 5), (7, 6))[y]
    return 8 * z + row[x]


def _hamiltonian_cycle():
    cyc = []
    for yi, y in enumerate(range(4)):
        zs = range(4) if yi % 2 == 0 else range(3, -1, -1)
        cyc.extend(_logical_id(0, y, z) for z in zs)
    for yi, y in enumerate(range(3, -1, -1)):
        zs = range(4) if yi % 2 == 0 else range(3, -1, -1)
        cyc.extend(_logical_id(1, y, z) for z in zs)
    assert sorted(cyc) == list(range(N_DEV))
    return np.array(cyc, dtype=np.int32)


_CYC = _hamiltonian_cycle()
_CPOS = np.argsort(_CYC).astype(np.int32)


def kernel(x, w_mat):
    m, k_loc = x.shape
    _, n = w_mat.shape
    m_per = m // N_DEV
    n_q = n // 4

    cyc = jnp.asarray(_CYC)
    q = jnp.asarray(_CPOS)[lax.axis_index("i")]
    js = jnp.arange(N_DEV, dtype=jnp.int32)
    nbrs = jnp.stack([cyc[(q + 1) % N_DEV],
                      cyc[(q - 1) % N_DEV]])
    cseq_r = cyc[(q - 1 - js) % N_DEV]
    cseq_l = cyc[(q + 1 + js) % N_DEV]

    m_half = m_per // 2

    def body(x_ref, w_ref, nbrs_ref, cr_ref, cl_ref, out_ref, *scr):
        sbuf = scr[0:8]
        rbuf = scr[8:16]
        ssem = scr[16:24]
        rsem = scr[24:32]
        cred = scr[32:40]

        right = nbrs_ref[0]
        left = nbrs_ref[1]

        def downstream(qi):
            return right if qi < 2 else left

        def upstream(qi):
            return left if qi < 2 else right

        def seq(qi):
            return cr_ref if qi < 2 else cl_ref

        barrier = pltpu.get_barrier_semaphore()
        pl.semaphore_signal(barrier, inc=1, device_id=(left,),
                            device_id_type=pl.DeviceIdType.MESH)
        pl.semaphore_signal(barrier, inc=1, device_id=(right,),
                            device_id_type=pl.DeviceIdType.MESH)
        pl.semaphore_wait(barrier, 2)

        def partial(qi, r, c):
            xs = x_ref[pl.ds(c * m_per + r * m_half, m_half), :]
            return jnp.dot(xs, w_ref[:, qi * n_q:(qi + 1) * n_q],
                           preferred_element_type=jnp.float32)

        def desc(sid, qi, slot):
            return pltpu.make_async_remote_copy(
                src_ref=sbuf[sid].at[slot], dst_ref=rbuf[sid].at[slot],
                send_sem=ssem[sid].at[slot], recv_sem=rsem[sid].at[slot],
                device_id=(downstream(qi),),
                device_id_type=pl.DeviceIdType.MESH,
            )

        for qi, r in ORDER:
            sid = qi * 2 + r
            sbuf[sid][0, :, :] = partial(qi, r, seq(qi)[0]).astype(
                jnp.bfloat16)
            desc(sid, qi, 0).start()

        for qi, r in ORDER:
            sid = qi * 2 + r
            p = partial(qi, r, seq(qi)[1])
            desc(sid, qi, 0).wait_recv()
            acc = rbuf[sid][0, :, :].astype(jnp.float32) + p
            sbuf[sid][1, :, :] = acc.astype(jnp.bfloat16)
            pl.semaphore_signal(cred[sid], inc=1, device_id=(upstream(qi),),
                                device_id_type=pl.DeviceIdType.MESH)
            desc(sid, qi, 1).start()

        def hop(h, carry):
            s = lax.rem(h, 2)
            s2 = lax.rem(h + 1, 2)
            for qi, r in ORDER:
                sid = qi * 2 + r
                c_in = seq(qi)[h + 1]
                p = partial(qi, r, c_in)
                desc(sid, qi, s).wait_recv()
                acc = rbuf[sid][s, :, :].astype(jnp.float32) + p
                desc(sid, qi, s2).wait_send()
                sbuf[sid][s2, :, :] = acc.astype(jnp.bfloat16)

                @pl.when(h <= N_DEV - 4)
                def _():
                    pl.semaphore_signal(
                        cred[sid], inc=1, device_id=(upstream(qi),),
                        device_id_type=pl.DeviceIdType.MESH)

                pl.semaphore_wait(cred[sid], 1)
                desc(sid, qi, s2).start()
            return carry

        lax.fori_loop(1, N_DEV - 2, hop, 0)

        for qi, r in ORDER:
            sid = qi * 2 + r
            p = partial(qi, r, seq(qi)[N_DEV - 1])
            desc(sid, qi, 0).wait_recv()
            acc = rbuf[sid][0, :, :].astype(jnp.float32) + p
            out_ref[r * m_half:(r + 1) * m_half,
                    qi * n_q:(qi + 1) * n_q] = jnp.maximum(acc, 0.0)
            desc(sid, qi, 1).wait_send()
            desc(sid, qi, 0).wait_send()

    buf = lambda: pltpu.VMEM((2, m // N_DEV // 2, n // 4), jnp.bfloat16)
    return pl.pallas_call(
        body,
        out_shape=jax.ShapeDtypeStruct((m_per, n), jnp.float32),
        in_specs=[
            pl.BlockSpec(memory_space=pltpu.VMEM),
            pl.BlockSpec(memory_space=pltpu.VMEM),
            pl.BlockSpec(memory_space=pltpu.SMEM),
            pl.BlockSpec(memory_space=pltpu.SMEM),
            pl.BlockSpec(memory_space=pltpu.SMEM),
        ],
        out_specs=pl.BlockSpec(memory_space=pltpu.VMEM),
        scratch_shapes=(
            [buf() for _ in range(8)]
            + [buf() for _ in range(8)]
            + [pltpu.SemaphoreType.DMA((2,)) for _ in range(8)]
            + [pltpu.SemaphoreType.DMA((2,)) for _ in range(8)]
            + [pltpu.SemaphoreType.REGULAR for _ in range(8)]
        ),
        compiler_params=pltpu.CompilerParams(collective_id=0),
    )(x, w_mat, nbrs, cseq_r, cseq_l)


# device time: 360576 ns/iter; 1.0204x vs baseline; 1.0204x over previous
import jax
import jax.numpy as jnp
from jax import lax
from jax.experimental import pallas as pl
from jax.experimental.pallas import tpu as pltpu

N_DEV = 32
ORDER = (0, 2, 1, 3)


def kernel(x, w_mat):
    m, k_loc = x.shape
    _, n = w_mat.shape
    m_per = m // N_DEV
    n_q = n // 4

    def body(x_ref, w_ref, out_ref, *scr):
        sbuf = scr[0:4]
        rbuf = scr[4:8]
        ssem = scr[8:12]
        rsem = scr[12:16]
        cred = scr[16:20]

        my = lax.axis_index("i")
        z_my = my // 8
        rp = my % 8
        y_my = rp // 2
        x_my = jnp.where(y_my % 2 == 0, rp % 2, 1 - rp % 2)
        qpos = jnp.where(
            x_my == 0,
            4 * y_my + jnp.where(y_my % 2 == 0, z_my, 3 - z_my),
            16 + 4 * (3 - y_my)
            + jnp.where((3 - y_my) % 2 == 0, z_my, 3 - z_my),
        )

        def pos_to_id(p):
            p = lax.rem(p + 2 * N_DEV, N_DEV)
            xx = p // 16
            pp = p % 16
            yy = pp // 4
            zz = pp % 4
            yv = jnp.where(xx == 0, yy, 3 - yy)
            zv = jnp.where(yy % 2 == 0, zz, 3 - zz)
            return 8 * zv + 2 * yv + jnp.where(yv % 2 == 0, xx, 1 - xx)

        right = pos_to_id(qpos + 1)
        left = pos_to_id(qpos - 1)

        def downstream(qi):
            return right if qi < 2 else left

        def upstream(qi):
            return left if qi < 2 else right

        def send_chunk(qi, j):
            return pos_to_id(jnp.where(qi < 2, qpos - 1 - j, qpos + 1 + j))

        barrier = pltpu.get_barrier_semaphore()
        pl.semaphore_signal(barrier, inc=1, device_id=(left,),
                            device_id_type=pl.DeviceIdType.MESH)
        pl.semaphore_signal(barrier, inc=1, device_id=(right,),
                            device_id_type=pl.DeviceIdType.MESH)
        pl.semaphore_wait(barrier, 2)

        def partial(qi, c):
            xs = x_ref[pl.ds(c * m_per, m_per), :]
            return jnp.dot(xs, w_ref[:, qi * n_q:(qi + 1) * n_q],
                           preferred_element_type=jnp.float32)

        def desc(qi, slot):
            return pltpu.make_async_remote_copy(
                src_ref=sbuf[qi].at[slot], dst_ref=rbuf[qi].at[slot],
                send_sem=ssem[qi].at[slot], recv_sem=rsem[qi].at[slot],
                device_id=(downstream(qi),),
                device_id_type=pl.DeviceIdType.MESH,
            )

        for qi in ORDER:
            sbuf[qi][0, :, :] = partial(qi, send_chunk(qi, 0)).astype(
                jnp.bfloat16)
            desc(qi, 0).start()

        for qi in ORDER:
            p = partial(qi, send_chunk(qi, 1))
            desc(qi, 0).wait_recv()
            acc = rbuf[qi][0, :, :].astype(jnp.float32) + p
            sbuf[qi][1, :, :] = acc.astype(jnp.bfloat16)
            pl.semaphore_signal(cred[qi], inc=1, device_id=(upstream(qi),),
                                device_id_type=pl.DeviceIdType.MESH)
            desc(qi, 1).start()

        def hop(h, carry):
            s = lax.rem(h, 2)
            s2 = lax.rem(h + 1, 2)
            for qi in ORDER:
                p = partial(qi, send_chunk(qi, h + 1))
                desc(qi, s).wait_recv()
                acc = rbuf[qi][s, :, :].astype(jnp.float32) + p
                desc(qi, s2).wait_send()
                sbuf[qi][s2, :, :] = acc.astype(jnp.bfloat16)

                @pl.when(h <= N_DEV - 4)
                def _():
                    pl.semaphore_signal(
                        cred[qi], inc=1, device_id=(upstream(qi),),
                        device_id_type=pl.DeviceIdType.MESH)

                pl.semaphore_wait(cred[qi], 1)
                desc(qi, s2).start()
            return carry

        lax.fori_loop(1, N_DEV - 2, hop, 0)

        for qi in ORDER:
            p = partial(qi, send_chunk(qi, N_DEV - 1))
            desc(qi, 0).wait_recv()
            acc = rbuf[qi][0, :, :].astype(jnp.float32) + p
            out_ref[:, qi * n_q:(qi + 1) * n_q] = jnp.maximum(acc, 0.0)
            desc(qi, 1).wait_send()
            desc(qi, 0).wait_send()

    buf = lambda: pltpu.VMEM((2, m // N_DEV, n // 4), jnp.bfloat16)
    return pl.pallas_call(
        body,
        out_shape=jax.ShapeDtypeStruct((m_per, n), jnp.float32),
        in_specs=[
            pl.BlockSpec(memory_space=pltpu.VMEM),
            pl.BlockSpec(memory_space=pltpu.VMEM),
        ],
        out_specs=pl.BlockSpec(memory_space=pltpu.VMEM),
        scratch_shapes=(
            [buf() for _ in range(4)]
            + [buf() for _ in range(4)]
            + [pltpu.SemaphoreType.DMA((2,)) for _ in range(4)]
            + [pltpu.SemaphoreType.DMA((2,)) for _ in range(4)]
            + [pltpu.SemaphoreType.REGULAR for _ in range(4)]
        ),
        compiler_params=pltpu.CompilerParams(collective_id=0),
    )(x, w_mat)
